# Initial kernel scaffold; baseline (speedup 1.0000x reference)
import jax
import jax.numpy as jnp
from jax.experimental import pallas as pl


def kernel(x, edge_index, batch, params):
    raise NotImplementedError("write your pallas kernel here")


# speedup vs baseline: 6.6742x; 6.6742x over previous
import functools

import jax
import jax.numpy as jnp
from jax import lax
from jax.experimental import pallas as pl
from jax.experimental.pallas import tpu as pltpu
from jax.experimental.pallas import tpu_sc as plsc

_NC = 2
_NS = 16
_NW = _NC * _NS

_IDX_B = 80
_SUP = 25


def _sc_scatter_add(src3d, dst3d, h):
    N, D = h.shape
    _, rows_per_tile, B = src3d.shape
    nlane = D // 16
    CH = -(-(N // _NS) // 8) * 8
    LAST = N - CH * (_NS - 1)
    assert LAST > 0

    mesh = plsc.VectorSubcoreMesh(core_axis_name="c", subcore_axis_name="s")

    @functools.partial(
        pl.kernel,
        out_type=jax.ShapeDtypeStruct((_NC, N, D), jnp.float32),
        mesh=mesh,
        scratch_types=[
            pltpu.VMEM((rows_per_tile, B), jnp.int32),
            pltpu.VMEM((rows_per_tile, B), jnp.int32),
            pltpu.VMEM((B, D), jnp.float32),
            pltpu.VMEM_SHARED((N, D), jnp.float32),
        ],
    )
    def k(src_hbm, dst_hbm, h_hbm, out_hbm, sidx, didx, rows, acc):
        c = lax.axis_index("c")
        s = lax.axis_index("s")
        wid = s * _NC + c

        def zrow(i, carry):
            for kk in range(nlane):
                rows[i, pl.ds(kk * 16, 16)] = jnp.zeros((16,), jnp.float32)
            return carry
        lax.fori_loop(0, B, zrow, 0)

        z0 = s * CH

        def zero_acc(nrows):
            nfull, rem = nrows // B, nrows % B

            def zcopy(i, carry):
                pltpu.sync_copy(rows, acc.at[pl.ds(z0 + i * B, B)])
                return carry
            lax.fori_loop(0, nfull, zcopy, 0)
            if rem:
                pltpu.sync_copy(rows.at[pl.ds(0, rem)],
                                acc.at[pl.ds(z0 + nfull * B, rem)])

        @pl.when(s < _NS - 1)
        def _zmost():
            zero_acc(CH)

        @pl.when(s == _NS - 1)
        def _zlast():
            zero_acc(LAST)

        plsc.subcore_barrier()

        pltpu.sync_copy(src_hbm.at[wid], sidx)
        pltpu.sync_copy(dst_hbm.at[wid], didx)

        def inner(j, carry):
            pltpu.sync_copy(h_hbm.at[sidx.at[j]], rows)
            pltpu.sync_copy(rows, acc.at[didx.at[j]], add=True)
            return carry
        lax.fori_loop(0, rows_per_tile, inner, 0)
        plsc.subcore_barrier()

        @pl.when(s < _NS - 1)
        def _cmost():
            pltpu.sync_copy(acc.at[pl.ds(z0, CH)],
                            out_hbm.at[c, pl.ds(z0, CH)])

        @pl.when(s == _NS - 1)
        def _clast():
            pltpu.sync_copy(acc.at[pl.ds(z0, LAST)],
                            out_hbm.at[c, pl.ds(z0, LAST)])

    return k(src3d, dst3d, h)


_BLK = 1000


def _gin_dense_body(h_ref, parts_ref, w1_ref, b1_ref, w2_ref, b2_ref,
                    g_ref, bb_ref, out_ref):
    x = h_ref[...] + parts_ref[0] + parts_ref[1]
    t = jnp.dot(x, w1_ref[...], preferred_element_type=jnp.float32)
    t = jnp.maximum(t + b1_ref[...], 0.0)
    t = jnp.dot(t, w2_ref[...], preferred_element_type=jnp.float32)
    t = t + b2_ref[...]
    mu = jnp.mean(t, axis=-1, keepdims=True)
    var = jnp.mean((t - mu) * (t - mu), axis=-1, keepdims=True)
    t = (t - mu) / jnp.sqrt(var + 1e-5) * g_ref[...] + bb_ref[...]
    out_ref[...] = jnp.maximum(t, 0.0)


def _gin_dense(h, parts, w1, b1, w2, b2, g, bb):
    N, D = h.shape
    grid = (N // _BLK,)
    return pl.pallas_call(
        _gin_dense_body,
        grid=grid,
        in_specs=[
            pl.BlockSpec((_BLK, D), lambda i: (i, 0)),
            pl.BlockSpec((_NC, _BLK, D), lambda i: (0, i, 0)),
            pl.BlockSpec((D, D), lambda i: (0, 0)),
            pl.BlockSpec((1, D), lambda i: (0, 0)),
            pl.BlockSpec((D, D), lambda i: (0, 0)),
            pl.BlockSpec((1, D), lambda i: (0, 0)),
            pl.BlockSpec((1, D), lambda i: (0, 0)),
            pl.BlockSpec((1, D), lambda i: (0, 0)),
        ],
        out_specs=pl.BlockSpec((_BLK, D), lambda i: (i, 0)),
        out_shape=jax.ShapeDtypeStruct((N, D), jnp.float32),
    )(h, parts, w1, b1, w2, b2, g, bb)


def _pool_mlp_body(h_ref, bf_ref, w1_ref, b1_ref, w2_ref, b2_ref,
                   out_ref, sums_ref, cnts_ref, *, nsteps, G):
    i = pl.program_id(0)

    @pl.when(i == 0)
    def _init():
        sums_ref[...] = jnp.zeros_like(sums_ref)
        cnts_ref[...] = jnp.zeros_like(cnts_ref)

    gids = lax.broadcasted_iota(jnp.int32, (bf_ref.shape[0], G), 1
                                ).astype(jnp.float32)
    onehot = (bf_ref[...] == gids).astype(jnp.float32)
    sums_ref[...] += lax.dot_general(
        onehot, h_ref[...], (((0,), (0,)), ((), ())),
        preferred_element_type=jnp.float32)
    cnts_ref[...] += jnp.sum(onehot, axis=0)[:, None]

    @pl.when(i == nsteps - 1)
    def _final():
        pooled = sums_ref[...] / jnp.maximum(cnts_ref[...], 1.0)
        z = jnp.dot(pooled, w1_ref[...], preferred_element_type=jnp.float32)
        z = jnp.maximum(z + b1_ref[...], 0.0)
        out_ref[...] = jnp.dot(z, w2_ref[...],
                               preferred_element_type=jnp.float32) + b2_ref[...]


def _pool_mlp(h, batch_f, w1, b1, w2, b2, G):
    N, D = h.shape
    nsteps = N // _BLK
    body = functools.partial(_pool_mlp_body, nsteps=nsteps, G=G)
    return pl.pallas_call(
        body,
        grid=(nsteps,),
        in_specs=[
            pl.BlockSpec((_BLK, D), lambda i: (i, 0)),
            pl.BlockSpec((_BLK, 1), lambda i: (i, 0)),
            pl.BlockSpec((D, D), lambda i: (0, 0)),
            pl.BlockSpec((1, D), lambda i: (0, 0)),
            pl.BlockSpec((D, 1), lambda i: (0, 0)),
            pl.BlockSpec((1, 1), lambda i: (0, 0)),
        ],
        out_specs=pl.BlockSpec((G, 1), lambda i: (0, 0)),
        out_shape=jax.ShapeDtypeStruct((G, 1), jnp.float32),
        scratch_shapes=[
            pltpu.VMEM((G, D), jnp.float32),
            pltpu.VMEM((G, 1), jnp.float32),
        ],
    )(h, batch_f, w1, b1, w2, b2)


def kernel(x, edge_index, batch, params):
    N, D = x.shape
    E = edge_index.shape[1]
    G = 64
    rows_per_tile = E // (_NW * _IDX_B)
    src3d = edge_index[0].reshape(_NW, rows_per_tile, _IDX_B)
    dst3d = edge_index[1].reshape(_NW, rows_per_tile, _IDX_B)
    batch_f = batch.astype(jnp.float32).reshape(N, 1)

    h = x
    for i in range(3):
        p = params['gin_%d' % i]
        parts = _sc_scatter_add(src3d, dst3d, h)
        h = _gin_dense(h, parts,
                       p['W1'], p['b1'].reshape(1, D),
                       p['W2'], p['b2'].reshape(1, D),
                       p['ln_g'].reshape(1, D), p['ln_b'].reshape(1, D))
    q = params['mlp']
    return _pool_mlp(h, batch_f,
                     q['W1'], q['b1'].reshape(1, D),
                     q['W2'], q['b2'].reshape(1, 1), G)


# speedup vs baseline: 8.6547x; 1.2967x over previous
import functools

import jax
import jax.numpy as jnp
from jax import lax
from jax.experimental import pallas as pl
from jax.experimental.pallas import tpu as pltpu
from jax.experimental.pallas import tpu_sc as plsc

_NC = 2
_NS = 16
_NW = _NC * _NS

_IDX_B = 80


def _sc_scatter_add(src3d, dst3d, h):
    N, D = h.shape
    _, rows_per_tile, B = src3d.shape
    nlane = D // 16
    CH = -(-(N // _NS) // 8) * 8
    LAST = N - CH * (_NS - 1)
    assert LAST > 0

    mesh = plsc.VectorSubcoreMesh(core_axis_name="c", subcore_axis_name="s")

    CHK = 40
    chunks = []
    q0 = 0
    while q0 < rows_per_tile:
        chunks.append((q0, min(CHK, rows_per_tile - q0)))
        q0 += CHK

    @functools.partial(
        pl.kernel,
        out_type=jax.ShapeDtypeStruct((_NC, N, D), jnp.float32),
        mesh=mesh,
        scratch_types=[
            pltpu.VMEM((CHK, B), jnp.int32),
            pltpu.VMEM((CHK, B), jnp.int32),
            pltpu.VMEM((CHK, B), jnp.int32),
            pltpu.VMEM((CHK, B), jnp.int32),
            pltpu.VMEM((B, D), jnp.float32),
            pltpu.VMEM((B, D), jnp.float32),
            pltpu.VMEM_SHARED((N, D), jnp.float32),
            pltpu.SemaphoreType.DMA,
            pltpu.SemaphoreType.DMA,
            pltpu.SemaphoreType.DMA,
            pltpu.SemaphoreType.DMA,
            pltpu.SemaphoreType.DMA,
        ],
    )
    def k(src_hbm, dst_hbm, h_hbm, out_hbm, sidxa, didxa, sidxb, didxb,
          rows0, rows1, acc, isem, gsem0, gsem1, ssem0, ssem1):
        c = lax.axis_index("c")
        s = lax.axis_index("s")
        wid = s * _NC + c

        ibufs = [(sidxa, didxa), (sidxb, didxb)]
        n0 = chunks[0][1]
        pltpu.async_copy(src_hbm.at[wid, pl.ds(0, n0)],
                         sidxa.at[pl.ds(0, n0)], isem)
        pltpu.async_copy(dst_hbm.at[wid, pl.ds(0, n0)],
                         didxa.at[pl.ds(0, n0)], isem)

        def zrow(i, carry):
            for kk in range(nlane):
                rows0[i, pl.ds(kk * 16, 16)] = jnp.zeros((16,), jnp.float32)
            return carry
        lax.fori_loop(0, B, zrow, 0)

        z0 = s * CH

        def zero_acc(nrows):
            nfull, rem = nrows // B, nrows % B

            def zcopy(i, carry):
                pltpu.sync_copy(rows0, acc.at[pl.ds(z0 + i * B, B)])
                return carry
            lax.fori_loop(0, nfull, zcopy, 0)
            if rem:
                pltpu.sync_copy(rows0.at[pl.ds(0, rem)],
                                acc.at[pl.ds(z0 + nfull * B, rem)])

        @pl.when(s < _NS - 1)
        def _zmost():
            zero_acc(CH)

        @pl.when(s == _NS - 1)
        def _zlast():
            zero_acc(LAST)

        pltpu.make_async_copy(src_hbm.at[wid, pl.ds(0, n0)],
                              sidxa.at[pl.ds(0, n0)], isem).wait()
        pltpu.make_async_copy(dst_hbm.at[wid, pl.ds(0, n0)],
                              didxa.at[pl.ds(0, n0)], isem).wait()
        plsc.subcore_barrier()

        for qi, (qs, qn) in enumerate(chunks):
            sidx, didx = ibufs[qi % 2]
            if qi + 1 < len(chunks):
                nqs, nqn = chunks[qi + 1]
                sb, db = ibufs[(qi + 1) % 2]
                pltpu.async_copy(src_hbm.at[wid, pl.ds(nqs, nqn)],
                                 sb.at[pl.ds(0, nqn)], isem)
                pltpu.async_copy(dst_hbm.at[wid, pl.ds(nqs, nqn)],
                                 db.at[pl.ds(0, nqn)], isem)
            npairs = qn // 2
            odd = qn % 2
            pltpu.async_copy(h_hbm.at[sidx.at[0]], rows0, gsem0)

            def body(u, carry, sidx=sidx, didx=didx, npairs=npairs, odd=odd):
                j0 = 2 * u
                j1 = j0 + 1
                pltpu.async_copy(h_hbm.at[sidx.at[j1]], rows1, gsem1)
                pltpu.make_async_copy(h_hbm.at[sidx.at[j0]], rows0,
                                      gsem0).wait()
                pltpu.async_copy(rows0, acc.at[didx.at[j0]], ssem0, add=True)
                pltpu.make_async_copy(h_hbm.at[sidx.at[j1]], rows1,
                                      gsem1).wait()
                pltpu.async_copy(rows1, acc.at[didx.at[j1]], ssem1, add=True)
                pltpu.make_async_copy(rows0, acc.at[didx.at[j0]],
                                      ssem0).wait()

                @pl.when((u + 1 < npairs) | (odd != 0))
                def _next():
                    pltpu.async_copy(h_hbm.at[sidx.at[j0 + 2]], rows0, gsem0)

                pltpu.make_async_copy(rows1, acc.at[didx.at[j1]],
                                      ssem1).wait()
                return carry
            lax.fori_loop(0, npairs, body, 0)
            if odd:
                j = qn - 1
                if npairs == 0:
                    pltpu.async_copy(h_hbm.at[sidx.at[j]], rows0, gsem0)
                pltpu.make_async_copy(h_hbm.at[sidx.at[j]], rows0,
                                      gsem0).wait()
                pltpu.sync_copy(rows0, acc.at[didx.at[j]], add=True)
            if qi + 1 < len(chunks):
                nqs, nqn = chunks[qi + 1]
                sb, db = ibufs[(qi + 1) % 2]
                pltpu.make_async_copy(src_hbm.at[wid, pl.ds(nqs, nqn)],
                                      sb.at[pl.ds(0, nqn)], isem).wait()
                pltpu.make_async_copy(dst_hbm.at[wid, pl.ds(nqs, nqn)],
                                      db.at[pl.ds(0, nqn)], isem).wait()
        plsc.subcore_barrier()

        @pl.when(s < _NS - 1)
        def _cmost():
            pltpu.sync_copy(acc.at[pl.ds(z0, CH)],
                            out_hbm.at[c, pl.ds(z0, CH)])

        @pl.when(s == _NS - 1)
        def _clast():
            pltpu.sync_copy(acc.at[pl.ds(z0, LAST)],
                            out_hbm.at[c, pl.ds(z0, LAST)])

    return k(src3d, dst3d, h)


_BLK = 1000


def _gin_dense_body(h_ref, parts_ref, w1_ref, b1_ref, w2_ref, b2_ref,
                    g_ref, bb_ref, out_ref):
    x = h_ref[...] + parts_ref[0] + parts_ref[1]
    t = jnp.dot(x, w1_ref[...], preferred_element_type=jnp.float32)
    t = jnp.maximum(t + b1_ref[...], 0.0)
    t = jnp.dot(t, w2_ref[...], preferred_element_type=jnp.float32)
    t = t + b2_ref[...]
    mu = jnp.mean(t, axis=-1, keepdims=True)
    var = jnp.mean((t - mu) * (t - mu), axis=-1, keepdims=True)
    t = (t - mu) / jnp.sqrt(var + 1e-5) * g_ref[...] + bb_ref[...]
    out_ref[...] = jnp.maximum(t, 0.0)


def _gin_dense(h, parts, w1, b1, w2, b2, g, bb):
    N, D = h.shape
    grid = (N // _BLK,)
    return pl.pallas_call(
        _gin_dense_body,
        grid=grid,
        in_specs=[
            pl.BlockSpec((_BLK, D), lambda i: (i, 0)),
            pl.BlockSpec((_NC, _BLK, D), lambda i: (0, i, 0)),
            pl.BlockSpec((D, D), lambda i: (0, 0)),
            pl.BlockSpec((1, D), lambda i: (0, 0)),
            pl.BlockSpec((D, D), lambda i: (0, 0)),
            pl.BlockSpec((1, D), lambda i: (0, 0)),
            pl.BlockSpec((1, D), lambda i: (0, 0)),
            pl.BlockSpec((1, D), lambda i: (0, 0)),
        ],
        out_specs=pl.BlockSpec((_BLK, D), lambda i: (i, 0)),
        out_shape=jax.ShapeDtypeStruct((N, D), jnp.float32),
    )(h, parts, w1, b1, w2, b2, g, bb)


def _pool_mlp_body(h_ref, bf_ref, w1_ref, b1_ref, w2_ref, b2_ref,
                   out_ref, sums_ref, cnts_ref, *, nsteps, G):
    i = pl.program_id(0)

    @pl.when(i == 0)
    def _init():
        sums_ref[...] = jnp.zeros_like(sums_ref)
        cnts_ref[...] = jnp.zeros_like(cnts_ref)

    gids = lax.broadcasted_iota(jnp.int32, (bf_ref.shape[0], G), 1
                                ).astype(jnp.float32)
    onehot = (bf_ref[...] == gids).astype(jnp.float32)
    sums_ref[...] += lax.dot_general(
        onehot, h_ref[...], (((0,), (0,)), ((), ())),
        preferred_element_type=jnp.float32)
    cnts_ref[...] += jnp.sum(onehot, axis=0)[:, None]

    @pl.when(i == nsteps - 1)
    def _final():
        pooled = sums_ref[...] / jnp.maximum(cnts_ref[...], 1.0)
        z = jnp.dot(pooled, w1_ref[...], preferred_element_type=jnp.float32)
        z = jnp.maximum(z + b1_ref[...], 0.0)
        out_ref[...] = jnp.dot(z, w2_ref[...],
                               preferred_element_type=jnp.float32) + b2_ref[...]


def _pool_mlp(h, batch_f, w1, b1, w2, b2, G):
    N, D = h.shape
    nsteps = N // _BLK
    body = functools.partial(_pool_mlp_body, nsteps=nsteps, G=G)
    return pl.pallas_call(
        body,
        grid=(nsteps,),
        in_specs=[
            pl.BlockSpec((_BLK, D), lambda i: (i, 0)),
            pl.BlockSpec((_BLK, 1), lambda i: (i, 0)),
            pl.BlockSpec((D, D), lambda i: (0, 0)),
            pl.BlockSpec((1, D), lambda i: (0, 0)),
            pl.BlockSpec((D, 1), lambda i: (0, 0)),
            pl.BlockSpec((1, 1), lambda i: (0, 0)),
        ],
        out_specs=pl.BlockSpec((G, 1), lambda i: (0, 0)),
        out_shape=jax.ShapeDtypeStruct((G, 1), jnp.float32),
        scratch_shapes=[
            pltpu.VMEM((G, D), jnp.float32),
            pltpu.VMEM((G, 1), jnp.float32),
        ],
    )(h, batch_f, w1, b1, w2, b2)


def kernel(x, edge_index, batch, params):
    N, D = x.shape
    E = edge_index.shape[1]
    G = 64
    rows_per_tile = E // (_NW * _IDX_B)
    src3d = edge_index[0].reshape(_NW, rows_per_tile, _IDX_B)
    dst3d = edge_index[1].reshape(_NW, rows_per_tile, _IDX_B)
    batch_f = batch.astype(jnp.float32).reshape(N, 1)

    h = x
    for i in range(3):
        p = params['gin_%d' % i]
        parts = _sc_scatter_add(src3d, dst3d, h)
        h = _gin_dense(h, parts,
                       p['W1'], p['b1'].reshape(1, D),
                       p['W2'], p['b2'].reshape(1, D),
                       p['ln_g'].reshape(1, D), p['ln_b'].reshape(1, D))
    q = params['mlp']
    return _pool_mlp(h, batch_f,
                     q['W1'], q['b1'].reshape(1, D),
                     q['W2'], q['b2'].reshape(1, 1), G)


# speedup vs baseline: 10.1117x; 1.1683x over previous
import functools

import jax
import jax.numpy as jnp
from jax import lax
from jax.experimental import pallas as pl
from jax.experimental.pallas import tpu as pltpu
from jax.experimental.pallas import tpu_sc as plsc

_NC = 2
_NS = 16
_NW = _NC * _NS

_IDX_B = 50
_NSLOT = 4
_CHK = 16


def _sc_scatter_add(src3d, dst3d, h):
    N, D = h.shape
    _, nb, B = src3d.shape
    nlane = D // 16
    chunks = []
    q0 = 0
    while q0 < nb:
        chunks.append((q0, min(_CHK, nb - q0)))
        q0 += _CHK
    assert all(qn % _NSLOT == 0 for _, qn in chunks)
    CH = -(-(N // _NS) // 8) * 8
    LAST = N - CH * (_NS - 1)
    assert LAST > 0

    mesh = plsc.VectorSubcoreMesh(core_axis_name="c", subcore_axis_name="s")

    @functools.partial(
        pl.kernel,
        out_type=jax.ShapeDtypeStruct((_NC, N, D), jnp.float32),
        mesh=mesh,
        scratch_types=[
            [pltpu.VMEM((_CHK, B), jnp.int32)] * 2,
            [pltpu.VMEM((_CHK, B), jnp.int32)] * 2,
            [pltpu.VMEM((B, D), jnp.float32)] * _NSLOT,
            pltpu.VMEM_SHARED((N, D), jnp.float32),
            pltpu.SemaphoreType.DMA,
            [pltpu.SemaphoreType.DMA] * _NSLOT,
            [pltpu.SemaphoreType.DMA] * _NSLOT,
        ],
    )
    def k(src_hbm, dst_hbm, h_hbm, out_hbm, sidxs, didxs, rows, acc,
          isem, gsems, ssems):
        c = lax.axis_index("c")
        s = lax.axis_index("s")
        wid = s * _NC + c

        n0 = chunks[0][1]
        pltpu.async_copy(src_hbm.at[wid, pl.ds(0, n0)],
                         sidxs[0].at[pl.ds(0, n0)], isem)
        pltpu.async_copy(dst_hbm.at[wid, pl.ds(0, n0)],
                         didxs[0].at[pl.ds(0, n0)], isem)

        def zrow(i, carry):
            for kk in range(nlane):
                rows[0][i, pl.ds(kk * 16, 16)] = jnp.zeros((16,), jnp.float32)
            return carry
        lax.fori_loop(0, B, zrow, 0)

        z0 = s * CH

        def zero_acc(nrows):
            nfull, rem = nrows // B, nrows % B

            def zcopy(i, carry):
                pltpu.sync_copy(rows[0], acc.at[pl.ds(z0 + i * B, B)])
                return carry
            lax.fori_loop(0, nfull, zcopy, 0)
            if rem:
                pltpu.sync_copy(rows[0].at[pl.ds(0, rem)],
                                acc.at[pl.ds(z0 + nfull * B, rem)])

        @pl.when(s < _NS - 1)
        def _zmost():
            zero_acc(CH)

        @pl.when(s == _NS - 1)
        def _zlast():
            zero_acc(LAST)

        pltpu.make_async_copy(src_hbm.at[wid, pl.ds(0, n0)],
                              sidxs[0].at[pl.ds(0, n0)], isem).wait()
        pltpu.make_async_copy(dst_hbm.at[wid, pl.ds(0, n0)],
                              didxs[0].at[pl.ds(0, n0)], isem).wait()
        plsc.subcore_barrier()

        for qi, (qs, qn) in enumerate(chunks):
            sidx, didx = sidxs[qi % 2], didxs[qi % 2]
            if qi + 1 < len(chunks):
                nqs, nqn = chunks[qi + 1]
                sb, db = sidxs[(qi + 1) % 2], didxs[(qi + 1) % 2]
                pltpu.async_copy(src_hbm.at[wid, pl.ds(nqs, nqn)],
                                 sb.at[pl.ds(0, nqn)], isem)
                pltpu.async_copy(dst_hbm.at[wid, pl.ds(nqs, nqn)],
                                 db.at[pl.ds(0, nqn)], isem)

            def gather(j, slot, sidx=sidx):
                pltpu.async_copy(h_hbm.at[sidx.at[j]], rows[slot],
                                 gsems[slot])

            def gwait(j, slot, sidx=sidx):
                pltpu.make_async_copy(h_hbm.at[sidx.at[j]], rows[slot],
                                      gsems[slot]).wait()

            def scatter(j, slot, didx=didx):
                pltpu.async_copy(rows[slot], acc.at[didx.at[j]],
                                 ssems[slot], add=True)

            def swait(j, slot, didx=didx):
                pltpu.make_async_copy(rows[slot], acc.at[didx.at[j]],
                                      ssems[slot]).wait()

            nr = qn // _NSLOT
            for k0 in range(_NSLOT):
                gather(k0, k0)

            def round_body(u, carry):
                j0 = u * _NSLOT
                for k0 in range(_NSLOT):
                    gwait(j0 + k0, k0)
                    scatter(j0 + k0, k0)
                for k0 in range(_NSLOT):
                    swait(j0 + k0, k0)

                    @pl.when(u + 1 < nr)
                    def _next(k0=k0, j0=j0):
                        gather(j0 + _NSLOT + k0, k0)
                return carry
            lax.fori_loop(0, nr, round_body, 0)

            if qi + 1 < len(chunks):
                nqs, nqn = chunks[qi + 1]
                sb, db = sidxs[(qi + 1) % 2], didxs[(qi + 1) % 2]
                pltpu.make_async_copy(src_hbm.at[wid, pl.ds(nqs, nqn)],
                                      sb.at[pl.ds(0, nqn)], isem).wait()
                pltpu.make_async_copy(dst_hbm.at[wid, pl.ds(nqs, nqn)],
                                      db.at[pl.ds(0, nqn)], isem).wait()
        plsc.subcore_barrier()

        @pl.when(s < _NS - 1)
        def _cmost():
            pltpu.sync_copy(acc.at[pl.ds(z0, CH)],
                            out_hbm.at[c, pl.ds(z0, CH)])

        @pl.when(s == _NS - 1)
        def _clast():
            pltpu.sync_copy(acc.at[pl.ds(z0, LAST)],
                            out_hbm.at[c, pl.ds(z0, LAST)])

    return k(src3d, dst3d, h)


_BLK = 1000


def _gin_dense_body(h_ref, parts_ref, w1_ref, b1_ref, w2_ref, b2_ref,
                    g_ref, bb_ref, out_ref):
    x = h_ref[...] + parts_ref[0] + parts_ref[1]
    t = jnp.dot(x, w1_ref[...], preferred_element_type=jnp.float32)
    t = jnp.maximum(t + b1_ref[...], 0.0)
    t = jnp.dot(t, w2_ref[...], preferred_element_type=jnp.float32)
    t = t + b2_ref[...]
    mu = jnp.mean(t, axis=-1, keepdims=True)
    var = jnp.mean((t - mu) * (t - mu), axis=-1, keepdims=True)
    t = (t - mu) / jnp.sqrt(var + 1e-5) * g_ref[...] + bb_ref[...]
    out_ref[...] = jnp.maximum(t, 0.0)


def _gin_dense(h, parts, w1, b1, w2, b2, g, bb):
    N, D = h.shape
    grid = (N // _BLK,)
    return pl.pallas_call(
        _gin_dense_body,
        grid=grid,
        in_specs=[
            pl.BlockSpec((_BLK, D), lambda i: (i, 0)),
            pl.BlockSpec((_NC, _BLK, D), lambda i: (0, i, 0)),
            pl.BlockSpec((D, D), lambda i: (0, 0)),
            pl.BlockSpec((1, D), lambda i: (0, 0)),
            pl.BlockSpec((D, D), lambda i: (0, 0)),
            pl.BlockSpec((1, D), lambda i: (0, 0)),
            pl.BlockSpec((1, D), lambda i: (0, 0)),
            pl.BlockSpec((1, D), lambda i: (0, 0)),
        ],
        out_specs=pl.BlockSpec((_BLK, D), lambda i: (i, 0)),
        out_shape=jax.ShapeDtypeStruct((N, D), jnp.float32),
    )(h, parts, w1, b1, w2, b2, g, bb)


def _pool_mlp_body(h_ref, bf_ref, w1_ref, b1_ref, w2_ref, b2_ref,
                   out_ref, sums_ref, cnts_ref, *, nsteps, G):
    i = pl.program_id(0)

    @pl.when(i == 0)
    def _init():
        sums_ref[...] = jnp.zeros_like(sums_ref)
        cnts_ref[...] = jnp.zeros_like(cnts_ref)

    gids = lax.broadcasted_iota(jnp.int32, (bf_ref.shape[0], G), 1
                                ).astype(jnp.float32)
    onehot = (bf_ref[...] == gids).astype(jnp.float32)
    sums_ref[...] += lax.dot_general(
        onehot, h_ref[...], (((0,), (0,)), ((), ())),
        preferred_element_type=jnp.float32)
    cnts_ref[...] += jnp.sum(onehot, axis=0)[:, None]

    @pl.when(i == nsteps - 1)
    def _final():
        pooled = sums_ref[...] / jnp.maximum(cnts_ref[...], 1.0)
        z = jnp.dot(pooled, w1_ref[...], preferred_element_type=jnp.float32)
        z = jnp.maximum(z + b1_ref[...], 0.0)
        out_ref[...] = jnp.dot(z, w2_ref[...],
                               preferred_element_type=jnp.float32) + b2_ref[...]


def _pool_mlp(h, batch_f, w1, b1, w2, b2, G):
    N, D = h.shape
    nsteps = N // _BLK
    body = functools.partial(_pool_mlp_body, nsteps=nsteps, G=G)
    return pl.pallas_call(
        body,
        grid=(nsteps,),
        in_specs=[
            pl.BlockSpec((_BLK, D), lambda i: (i, 0)),
            pl.BlockSpec((_BLK, 1), lambda i: (i, 0)),
            pl.BlockSpec((D, D), lambda i: (0, 0)),
            pl.BlockSpec((1, D), lambda i: (0, 0)),
            pl.BlockSpec((D, 1), lambda i: (0, 0)),
            pl.BlockSpec((1, 1), lambda i: (0, 0)),
        ],
        out_specs=pl.BlockSpec((G, 1), lambda i: (0, 0)),
        out_shape=jax.ShapeDtypeStruct((G, 1), jnp.float32),
        scratch_shapes=[
            pltpu.VMEM((G, D), jnp.float32),
            pltpu.VMEM((G, 1), jnp.float32),
        ],
    )(h, batch_f, w1, b1, w2, b2)


def kernel(x, edge_index, batch, params):
    N, D = x.shape
    E = edge_index.shape[1]
    G = 64
    nb = E // (_NW * _IDX_B)
    src3d = edge_index[0].reshape(_NW, nb, _IDX_B)
    dst3d = edge_index[1].reshape(_NW, nb, _IDX_B)
    batch_f = batch.astype(jnp.float32).reshape(N, 1)

    h = x
    for i in range(3):
        p = params['gin_%d' % i]
        parts = _sc_scatter_add(src3d, dst3d, h)
        h = _gin_dense(h, parts,
                       p['W1'], p['b1'].reshape(1, D),
                       p['W2'], p['b2'].reshape(1, D),
                       p['ln_g'].reshape(1, D), p['ln_b'].reshape(1, D))
    q = params['mlp']
    return _pool_mlp(h, batch_f,
                     q['W1'], q['b1'].reshape(1, D),
                     q['W2'], q['b2'].reshape(1, 1), G)
